# baseline (speedup 1.0000x reference)
import functools

import jax
import jax.numpy as jnp
import numpy as np
from jax import lax
from jax.experimental import pallas as pl
from jax.experimental.pallas import tpu as pltpu
from jax.experimental.pallas import tpu_sc as plsc

VOCAB = 1000000
DIM = 128
B = 16384
L = 50
N = B * L

NC = 2
NS = 16
NW = NC * NS
CHUNK = 64
NSLOT = 3
NPART = 4
NP = N // NPART
PER_W = NP // NW
STEPS = PER_W // CHUNK


def _make_sc_body(part):
    base = part * NP

    def _sc_body(x_hbm, e_hbm, node_hbm, edge_hbm, out_hbm,
                 xidx, eidx, nbuf, ebuf, hbuf, *sems):
        wid = lax.axis_index("s") * NC + lax.axis_index("c")
        gn = sems[0:NSLOT]
        ge = sems[NSLOT:2 * NSLOT]
        ss = sems[2 * NSLOT:3 * NSLOT]
        pltpu.sync_copy(x_hbm.at[pl.ds(base + wid * PER_W, PER_W)], xidx)
        pltpu.sync_copy(e_hbm.at[pl.ds(base + wid * PER_W, PER_W)], eidx)

        def start_gather(i, b):
            s = pl.ds(i * CHUNK, CHUNK)
            pltpu.async_copy(node_hbm.at[xidx.at[s]], nbuf.at[b], gn[b])
            pltpu.async_copy(edge_hbm.at[eidx.at[s]], ebuf.at[b], ge[b])

        def wait_gather(i, b):
            s = pl.ds(i * CHUNK, CHUNK)
            pltpu.make_async_copy(node_hbm.at[xidx.at[s]], nbuf.at[b], gn[b]).wait()
            pltpu.make_async_copy(edge_hbm.at[eidx.at[s]], ebuf.at[b], ge[b]).wait()

        def out_slice(i):
            return out_hbm.at[pl.ds((wid * STEPS + i) * CHUNK, CHUNK)]

        def wait_store(i, b):
            pltpu.make_async_copy(hbuf.at[b], out_slice(i), ss[b]).wait()

        def add_and_store(i, b):
            def add_row(r, _):
                for g in range(DIM // 32):
                    s0 = pl.ds(g * 32, 16)
                    s1 = pl.ds(g * 32 + 16, 16)
                    lo = nbuf[b, r, s0] + ebuf[b, r, s0]
                    hi = nbuf[b, r, s1] + ebuf[b, r, s1]
                    li = lax.bitcast_convert_type(lo, jnp.int32) + 0x8000
                    hii = lax.bitcast_convert_type(hi, jnp.int32) + 0x8000
                    w = lax.shift_right_logical(li, 16) | (hii & jnp.int32(-65536))
                    hbuf[b, r, pl.ds(g * 16, 16)] = w
                return 0

            lax.fori_loop(0, CHUNK, add_row, 0)
            pltpu.async_copy(hbuf.at[b], out_slice(i), ss[b])

        def do_step(i, b):
            nb = (b + 2) % NSLOT
            wait_store(i - 1, nb)
            start_gather(i + 2, nb)
            wait_gather(i, b)
            add_and_store(i, b)

        start_gather(0, 0)
        start_gather(1, 1)
        start_gather(2, 2)
        wait_gather(0, 0)
        add_and_store(0, 0)
        wait_store(0, 0)
        start_gather(3, 0)
        wait_gather(1, 1)
        add_and_store(1, 1)

        def triple(j, _):
            i = 2 + 3 * j
            do_step(i, 2)
            do_step(i + 1, 0)
            do_step(i + 2, 1)
            return 0

        lax.fori_loop(0, (STEPS - 4) // 3, triple, 0)

        wait_gather(STEPS - 2, (STEPS - 2) % NSLOT)
        add_and_store(STEPS - 2, (STEPS - 2) % NSLOT)
        wait_gather(STEPS - 1, (STEPS - 1) % NSLOT)
        add_and_store(STEPS - 1, (STEPS - 1) % NSLOT)
        for k in range(3):
            wait_store(STEPS - 3 + k, (STEPS - 3 + k) % NSLOT)

    return _sc_body


def _sc_gather_add(part, xf, ef, node_table, edge_table):
    mesh = plsc.VectorSubcoreMesh(core_axis_name="c", subcore_axis_name="s",
                                  num_cores=NC)
    k = pl.kernel(
        _make_sc_body(part),
        out_type=jax.ShapeDtypeStruct((NP, DIM // 2), jnp.int32),
        mesh=mesh,
        scratch_types=[
            pltpu.VMEM((PER_W,), jnp.int32),
            pltpu.VMEM((PER_W,), jnp.int32),
            pltpu.VMEM((NSLOT, CHUNK, DIM), jnp.float32),
            pltpu.VMEM((NSLOT, CHUNK, DIM), jnp.float32),
            pltpu.VMEM((NSLOT, CHUNK, DIM // 2), jnp.int32),
        ] + [pltpu.SemaphoreType.DMA] * (3 * NSLOT),
    )
    return k(xf, ef, node_table, edge_table)


def _leaky(v):
    return jnp.maximum(v, 0.1 * v)


def _mlp_body(h_ref, w1_ref, b1_ref, w2_ref, b2_ref, w3_ref, b3_ref,
              w4_ref, b4_ref, out_ref):
    def lin(v, w_ref, b_ref):
        return jnp.dot(v, w_ref[...], preferred_element_type=jnp.float32) + b_ref[...]

    h = _leaky(lin(h_ref[...], w1_ref, b1_ref))
    h = _leaky(lin(h, w2_ref, b2_ref))
    h = _leaky(lin(h, w3_ref, b3_ref))
    v = jnp.sum(h * w4_ref[...], axis=1) + b4_ref[0, 0]
    out_ref[...] = v.reshape(out_ref.shape)


def _mlp(h, w1t, b1, w2t, b2, w3t, b3, w4, b4, rows_per_block=8192):
    rows = h.shape[0]
    grid = (rows // rows_per_block,)
    wspec = pl.BlockSpec((DIM, DIM), lambda i: (0, 0))
    bspec = pl.BlockSpec((1, DIM), lambda i: (0, 0))
    return pl.pallas_call(
        _mlp_body,
        grid=grid,
        in_specs=[
            pl.BlockSpec((rows_per_block, DIM), lambda i: (i, 0)),
            wspec, bspec, wspec, bspec, wspec, bspec,
            bspec,
            pl.BlockSpec((1, 1), lambda i: (0, 0)),
        ],
        out_specs=pl.BlockSpec((rows_per_block // DIM, DIM), lambda i: (i, 0)),
        out_shape=jax.ShapeDtypeStruct((rows // DIM, DIM), jnp.float32),
    )(h, w1t, b1, w2t, b2, w3t, b3, w4, b4)


def kernel(x, edge_attr, node_table, edge_table, W1, b1, W2, b2, W3, b3, W4, b4):
    xf = x.astype(jnp.int32).reshape(N)
    ef = edge_attr.astype(jnp.int32).reshape(N)
    q = np.arange(DIM)
    perm = (q // 32) * 32 + (q % 2) * 16 + (q % 32) // 2
    w1p = W1.T[perm].astype(jnp.bfloat16)
    parts = []
    for k in range(NPART):
        h32 = _sc_gather_add(k, xf, ef, node_table, edge_table)
        h = jax.lax.bitcast_convert_type(h32, jnp.bfloat16).reshape(NP, DIM)
        parts.append(_mlp(h, w1p, b1.reshape(1, DIM), W2.T, b2.reshape(1, DIM),
                          W3.T, b3.reshape(1, DIM), W4, b4.reshape(1, 1)))
    logits = jnp.concatenate(parts, axis=0)
    return logits.reshape(B, L, 1)
